# Initial kernel scaffold; baseline (speedup 1.0000x reference)
import jax
import jax.numpy as jnp
from jax.experimental import pallas as pl


def kernel(grid_feat, ref_feat, grid_index, grid_xy):
    raise NotImplementedError("write your pallas kernel here")


# speedup vs baseline: 1.0266x; 1.0266x over previous
import functools

import jax
import jax.numpy as jnp
from jax import lax
from jax.experimental import pallas as pl
from jax.experimental.pallas import tpu as pltpu
from jax.experimental.pallas import tpu_sc as plsc

Bn = 4
Cn = 96
PHn = 96
PWn = 384
Hn = 384
Wn = 384
Nn = PHn * PWn
BN = Bn * Nn
NC = 2
NS = 16
Ln = 16
NW = NC * NS
RPW = BN // NW
Kc = 64
NCHUNK = RPW // Kc


def _t_body(x_ref, o_ref):
    o_ref[0] = x_ref[0].T


def _to_table(gf):
    HW = Hn * Wn
    T = 512
    out = pl.pallas_call(
        _t_body,
        grid=(Bn, HW // T),
        in_specs=[pl.BlockSpec((1, Cn, T), lambda b, t: (b, 0, t))],
        out_specs=pl.BlockSpec((1, T, Cn), lambda b, t: (b, t, 0)),
        out_shape=jax.ShapeDtypeStruct((Bn, HW, Cn), jnp.float32),
    )(gf)
    return out.reshape(Bn * HW, Cn)


def _from_rows(vals):
    T = 512
    out = pl.pallas_call(
        _t_body,
        grid=(Bn, Nn // T),
        in_specs=[pl.BlockSpec((1, T, Cn), lambda b, t: (b, t, 0))],
        out_specs=pl.BlockSpec((1, Cn, T), lambda b, t: (b, 0, t)),
        out_shape=jax.ShapeDtypeStruct((Bn, Cn, Nn), jnp.float32),
    )(vals.reshape(Bn, Nn, Cn))
    return out.reshape(Bn, Cn, PHn, PWn)


def _floor_i32(x):
    t = x.astype(jnp.int32)
    return jnp.where(x < t.astype(jnp.float32), t - 1, t)


def _sc_sample(table, gix, giy, oxb, oyy, oxx):
    mesh = plsc.VectorSubcoreMesh(
        core_axis_name="c", subcore_axis_name="s", num_cores=NC,
        num_subcores=NS)

    @functools.partial(
        pl.kernel,
        out_type=jax.ShapeDtypeStruct((BN, Cn), jnp.float32),
        mesh=mesh,
        compiler_params=pltpu.CompilerParams(
            needs_layout_passes=False, use_tc_tiling_on_sc=False),
        scratch_types=[
            pltpu.VMEM((RPW,), jnp.float32),
            pltpu.VMEM((RPW,), jnp.float32),
            pltpu.VMEM((RPW,), jnp.int32),
            pltpu.VMEM((RPW,), jnp.int32),
            pltpu.VMEM((RPW,), jnp.int32),
            pltpu.VMEM((2, 128), jnp.int32),
            pltpu.VMEM((4 * Kc,), jnp.float32),
            pltpu.VMEM((2 * Kc, Cn), jnp.float32),
            pltpu.VMEM((2 * Kc, Cn), jnp.float32),
            pltpu.VMEM((Kc, Cn), jnp.float32),
            pltpu.VMEM((Kc,), jnp.int32),
            pltpu.SemaphoreType.DMA,
            pltpu.SemaphoreType.DMA,
        ],
    )
    def sc_kernel(table_h, gix_h, giy_h, oxb_h, oyy_h, oxx_h, out_h,
                  gix_a, giy_a, ob_a, oy_a, ox_a, idx_v, w_v,
                  gbuf0, gbuf1, out_v, oidx_v, sem_g, sem_s):
        cid = lax.axis_index("c")
        sid = lax.axis_index("s")
        wid = sid * NC + cid
        base = wid * RPW
        bhw = (base // Nn) * (Hn * Wn)

        pltpu.sync_copy(gix_h.at[pl.ds(base, RPW)], gix_a)
        pltpu.sync_copy(giy_h.at[pl.ds(base, RPW)], giy_a)
        pltpu.sync_copy(oxb_h.at[pl.ds(base, RPW)], ob_a)
        pltpu.sync_copy(oyy_h.at[pl.ds(base, RPW)], oy_a)
        pltpu.sync_copy(oxx_h.at[pl.ds(base, RPW)], ox_a)

        def chunk(i, _):
            c0 = i * Kc
            for g in range(Kc // Ln):
                s = c0 + g * Ln
                gx = gix_a[pl.ds(s, Ln)]
                gy = giy_a[pl.ds(s, Ln)]
                x = (gx + 1.0) * ((Wn - 1) / 2.0)
                y = (gy + 1.0) * ((Hn - 1) / 2.0)
                x0 = _floor_i32(x)
                y0 = _floor_i32(y)
                x1 = x0 + 1
                y1 = y0 + 1
                wx1 = x - x0.astype(jnp.float32)
                wx0 = 1.0 - wx1
                wy1 = y - y0.astype(jnp.float32)
                wy0 = 1.0 - wy1
                inx0 = jnp.where((x0 >= 0) & (x0 <= Wn - 1), 1.0, 0.0)
                inx1 = jnp.where((x1 >= 0) & (x1 <= Wn - 1), 1.0, 0.0)
                iny0 = jnp.where((y0 >= 0) & (y0 <= Hn - 1), 1.0, 0.0)
                iny1 = jnp.where((y1 >= 0) & (y1 <= Hn - 1), 1.0, 0.0)
                x0c = jnp.minimum(jnp.maximum(x0, 0), Wn - 1)
                x1c = jnp.minimum(jnp.maximum(x1, 0), Wn - 1)
                y0c = jnp.minimum(jnp.maximum(y0, 0), Hn - 1)
                y1c = jnp.minimum(jnp.maximum(y1, 0), Hn - 1)
                rows = (bhw + y0c * Wn + x0c, bhw + y0c * Wn + x1c,
                        bhw + y1c * Wn + x0c, bhw + y1c * Wn + x1c)
                wts = (wx0 * wy0 * inx0 * iny0, wx1 * wy0 * inx1 * iny0,
                       wx0 * wy1 * inx0 * iny1, wx1 * wy1 * inx1 * iny1)
                ob = ob_a[pl.ds(s, Ln)]
                oy = oy_a[pl.ds(s, Ln)]
                ox = ox_a[pl.ds(s, Ln)]
                oidx_v[pl.ds(g * Ln, Ln)] = ob * Nn + oy * PWn + ox
                for cr in range(4):
                    flat = cr * Kc + g * Ln
                    idx_v[flat // 128, pl.ds(flat % 128, Ln)] = rows[cr]
                    w_v[pl.ds(flat, Ln)] = wts[cr]

            cp0 = pltpu.make_async_copy(table_h.at[idx_v.at[0]], gbuf0, sem_g)
            cp1 = pltpu.make_async_copy(table_h.at[idx_v.at[1]], gbuf1, sem_g)
            cp0.start()
            cp1.start()
            cp0.wait()
            cp1.wait()

            @plsc.parallel_loop(0, Kc, unroll=2)
            def _(p):
                ws0 = plsc.load_gather(w_v, [jnp.full((Ln,), p, jnp.int32)])
                ws1 = plsc.load_gather(
                    w_v, [jnp.full((Ln,), Kc + p, jnp.int32)])
                ws2 = plsc.load_gather(
                    w_v, [jnp.full((Ln,), 2 * Kc + p, jnp.int32)])
                ws3 = plsc.load_gather(
                    w_v, [jnp.full((Ln,), 3 * Kc + p, jnp.int32)])
                pk = Kc + p
                for cg in range(Cn // Ln):
                    sl = pl.ds(cg * Ln, Ln)
                    a = gbuf0[p, sl] * ws0
                    a = a + gbuf0[pk, sl] * ws1
                    a = a + gbuf1[p, sl] * ws2
                    a = a + gbuf1[pk, sl] * ws3
                    out_v[p, sl] = a

            pltpu.make_async_copy(out_v, out_h.at[oidx_v], sem_s).start()
            pltpu.make_async_copy(out_v, out_h.at[oidx_v], sem_s).wait()
            return 0

        lax.fori_loop(0, NCHUNK, chunk, 0)

    return sc_kernel(table, gix, giy, oxb, oyy, oxx)


def kernel(grid_feat, ref_feat, grid_index, grid_xy):
    del ref_feat
    table = _to_table(grid_feat.reshape(Bn, Cn, Hn * Wn))
    gi = grid_index.reshape(BN, 2)
    gxy = grid_xy.reshape(BN, 3)
    vals = _sc_sample(table, gi[:, 0], gi[:, 1],
                      gxy[:, 0], gxy[:, 1], gxy[:, 2])
    return _from_rows(vals)
